# baseline (speedup 1.0000x reference)
import functools

import jax
import jax.numpy as jnp
from jax import lax
from jax.experimental import pallas as pl
from jax.experimental.pallas import tpu as pltpu
from jax.experimental.pallas import tpu_sc as plsc

_CH = 128


@functools.cache
def _make_kernel(B, D, L, NC, NS):
    NW = NC * NS
    b_per_w = B // NW
    NCH = b_per_w // _CH
    JU = D // L
    mesh = plsc.VectorSubcoreMesh(core_axis_name="c", subcore_axis_name="s")

    @functools.partial(
        pl.kernel,
        mesh=mesh,
        out_type=jax.ShapeDtypeStruct((NW, L), jnp.float32),
        scratch_types=[
            pltpu.VMEM((_CH,), jnp.int32),
            pltpu.VMEM((_CH, D), jnp.float32),
            pltpu.VMEM((_CH, D), jnp.float32),
            pltpu.VMEM((L,), jnp.float32),
            pltpu.SemaphoreType.DMA,
        ],
    )
    def k(label_hbm, feat_hbm, centers_hbm, out_hbm, idx_v, rows_v, feat_v,
          part_v, sem):
        wid = lax.axis_index("s") * NC + lax.axis_index("c")
        base = wid * b_per_w

        accs = tuple(jnp.zeros((L,), jnp.float32) for _ in range(JU))
        for c in range(NCH):
            off = base + c * _CH
            pltpu.sync_copy(label_hbm.at[pl.ds(off, _CH)], idx_v)
            pltpu.async_copy(centers_hbm.at[idx_v], rows_v, sem).wait()
            pltpu.sync_copy(feat_hbm.at[pl.ds(off, _CH)], feat_v)

            def row_body(i, accs):
                new = []
                for j in range(JU):
                    f = feat_v[i, pl.ds(j * L, L)]
                    r = rows_v[i, pl.ds(j * L, L)]
                    d = f - r
                    new.append(accs[j] + d * d)
                return tuple(new)

            accs = lax.fori_loop(0, _CH, row_body, accs)

        tot = accs[0]
        for j in range(1, JU):
            tot = tot + accs[j]
        part_v[...] = tot
        pltpu.sync_copy(part_v, out_hbm.at[wid])

    return k


def kernel(label, feat, centers):
    B, D = feat.shape
    info = plsc.get_sparse_core_info()
    k = _make_kernel(B, D, info.num_lanes, info.num_cores, info.num_subcores)
    partials = k(label, feat, centers)
    return jnp.sum(partials) / (2.0 * B)


# speedup vs baseline: 1.2045x; 1.2045x over previous
import functools

import jax
import jax.numpy as jnp
from jax import lax
from jax.experimental import pallas as pl
from jax.experimental.pallas import tpu as pltpu
from jax.experimental.pallas import tpu_sc as plsc

_CH = 128


@functools.cache
def _make_kernel(B, D, L, NC, NS):
    NW = NC * NS
    b_per_w = B // NW
    NCH = b_per_w // _CH
    JU = D // L
    mesh = plsc.VectorSubcoreMesh(core_axis_name="c", subcore_axis_name="s")

    @functools.partial(
        pl.kernel,
        mesh=mesh,
        out_type=jax.ShapeDtypeStruct((NW, L), jnp.float32),
        scratch_types=[
            pltpu.VMEM((b_per_w,), jnp.int32),
            pltpu.VMEM((2, _CH, D), jnp.float32),
            pltpu.VMEM((b_per_w, D), jnp.float32),
            pltpu.VMEM((L,), jnp.float32),
            pltpu.SemaphoreType.DMA,
            pltpu.SemaphoreType.DMA,
            pltpu.SemaphoreType.DMA,
        ],
    )
    def k(label_hbm, feat_hbm, centers_hbm, out_hbm, idx_v, rows_v, feat_v,
          part_v, sem_f, sem_g0, sem_g1):
        wid = lax.axis_index("s") * NC + lax.axis_index("c")
        base = wid * b_per_w
        gsems = (sem_g0, sem_g1)

        pltpu.sync_copy(label_hbm.at[pl.ds(base, b_per_w)], idx_v)
        feat_cp = pltpu.async_copy(feat_hbm.at[pl.ds(base, b_per_w)], feat_v,
                                   sem_f)
        gathers = [pltpu.async_copy(
            centers_hbm.at[idx_v.at[pl.ds(0, _CH)]], rows_v.at[0], gsems[0])]

        accs = tuple(jnp.zeros((L,), jnp.float32) for _ in range(JU))
        for c in range(NCH):
            if c + 1 < NCH:
                nb = (c + 1) % 2
                gathers.append(pltpu.async_copy(
                    centers_hbm.at[idx_v.at[pl.ds((c + 1) * _CH, _CH)]],
                    rows_v.at[nb], gsems[nb]))
            gathers[c].wait()
            if c == 0:
                feat_cp.wait()
            buf = c % 2
            frow = c * _CH

            def row_body(i, accs, buf=buf, frow=frow):
                new = []
                for j in range(JU):
                    f = feat_v[frow + i, pl.ds(j * L, L)]
                    r = rows_v[buf, i, pl.ds(j * L, L)]
                    d = f - r
                    new.append(accs[j] + d * d)
                return tuple(new)

            accs = lax.fori_loop(0, _CH, row_body, accs)

        tot = accs[0]
        for j in range(1, JU):
            tot = tot + accs[j]
        part_v[...] = tot
        pltpu.sync_copy(part_v, out_hbm.at[wid])

    return k


def kernel(label, feat, centers):
    B, D = feat.shape
    info = plsc.get_sparse_core_info()
    k = _make_kernel(B, D, info.num_lanes, info.num_cores, info.num_subcores)
    partials = k(label, feat, centers)
    return jnp.sum(partials) / (2.0 * B)


# speedup vs baseline: 1.2144x; 1.0082x over previous
import functools

import jax
import jax.numpy as jnp
from jax import lax
from jax.experimental import pallas as pl
from jax.experimental.pallas import tpu as pltpu
from jax.experimental.pallas import tpu_sc as plsc

_CH = 128


@functools.cache
def _make_kernel(B, D, L, NC, NS):
    NW = NC * NS
    b_per_w = B // NW
    NCH = b_per_w // _CH
    JU = D // L
    mesh = plsc.VectorSubcoreMesh(core_axis_name="c", subcore_axis_name="s")

    @functools.partial(
        pl.kernel,
        mesh=mesh,
        out_type=jax.ShapeDtypeStruct((NW, L), jnp.float32),
        scratch_types=[
            pltpu.VMEM((b_per_w,), jnp.int32),
            pltpu.VMEM((2, _CH, D), jnp.float32),
            pltpu.VMEM((2, _CH, D), jnp.float32),
            pltpu.VMEM((L,), jnp.float32),
            pltpu.SemaphoreType.DMA,
            pltpu.SemaphoreType.DMA,
            pltpu.SemaphoreType.DMA,
            pltpu.SemaphoreType.DMA,
        ],
    )
    def k(label_hbm, feat_hbm, centers_hbm, out_hbm, idx_v, rows_v, feat_v,
          part_v, sem_f0, sem_f1, sem_g0, sem_g1):
        wid = lax.axis_index("s") * NC + lax.axis_index("c")
        base = wid * b_per_w
        gsems = (sem_g0, sem_g1)
        fsems = (sem_f0, sem_f1)

        def start(c):
            buf = c % 2
            g = pltpu.async_copy(
                centers_hbm.at[idx_v.at[pl.ds(c * _CH, _CH)]],
                rows_v.at[buf], gsems[buf])
            f = pltpu.async_copy(
                feat_hbm.at[pl.ds(base + c * _CH, _CH)],
                feat_v.at[buf], fsems[buf])
            return (g, f)

        pltpu.sync_copy(label_hbm.at[pl.ds(base, b_per_w)], idx_v)
        cps = [start(0)]

        accs = tuple(jnp.zeros((L,), jnp.float32) for _ in range(2 * JU))
        for c in range(NCH):
            if c + 1 < NCH:
                cps.append(start(c + 1))
            cps[c][0].wait()
            cps[c][1].wait()
            buf = c % 2

            def row_body(i, accs, buf=buf):
                i2 = i * 2
                new = []
                for u in range(2):
                    for j in range(JU):
                        f = feat_v[buf, i2 + u, pl.ds(j * L, L)]
                        r = rows_v[buf, i2 + u, pl.ds(j * L, L)]
                        d = f - r
                        new.append(accs[u * JU + j] + d * d)
                return tuple(new)

            accs = lax.fori_loop(0, _CH // 2, row_body, accs)

        tot = accs[0]
        for j in range(1, 2 * JU):
            tot = tot + accs[j]
        part_v[...] = tot
        pltpu.sync_copy(part_v, out_hbm.at[wid])

    return k


def kernel(label, feat, centers):
    B, D = feat.shape
    info = plsc.get_sparse_core_info()
    k = _make_kernel(B, D, info.num_lanes, info.num_cores, info.num_subcores)
    partials = k(label, feat, centers)
    return jnp.sum(partials) / (2.0 * B)
